# baseline (speedup 1.0000x reference)
import functools

import jax
import jax.numpy as jnp
from jax import lax
from jax.experimental import pallas as pl
from jax.experimental.pallas import tpu as pltpu
from jax.experimental.pallas import tpu_sc as plsc

N = 10000
E = 320000
D = 128
K = 10
ALPHA = 0.1

NC = 2
NS = 16
DH = D // NC
CH = 128
RING = 4
SW = 2
NCHUNK = 160
EPT = NCHUNK * CH
RPT = N // NS
RCH = 125
NRCH = RPT // RCH
LANES = 16
NBLK = NCHUNK // RING


_SCALES = tuple(ALPHA / (1.0 - ALPHA) ** (k + 1) for k in range(K))


def _linear_body(x_ref, w_ref, b_ref, o_ref, hs_ref):
    hb = lax.dot_general(
        x_ref[...], w_ref[...], (((1,), (1,)), ((), ())),
        preferred_element_type=jnp.float32) + b_ref[...]
    o_ref[...] = hb
    for k in range(K):
        hs_ref[k] = hb * _SCALES[k]


def _linear(x, W, b2):
    return pl.pallas_call(
        _linear_body,
        grid=(N // 1000,),
        in_specs=[pl.BlockSpec((1000, D), lambda i: (i, 0)),
                  pl.BlockSpec((D, D), lambda i: (0, 0)),
                  pl.BlockSpec((1, D), lambda i: (0, 0))],
        out_specs=[pl.BlockSpec((1000, D), lambda i: (i, 0)),
                   pl.BlockSpec((K, 1000, D), lambda i: (0, i, 0))],
        out_shape=[jax.ShapeDtypeStruct((N, D), jnp.float32),
                   jax.ShapeDtypeStruct((K, N, D), jnp.float32)],
    )(x, W, b2)


_FINAL_SCALE = (1.0 - ALPHA) ** K


def _lsm_body(v_ref, o_ref):
    z = v_ref[...] * _FINAL_SCALE
    m = jnp.max(z, axis=1, keepdims=True)
    zs = z - m
    o_ref[...] = zs - jnp.log(jnp.sum(jnp.exp(zs), axis=1, keepdims=True))


def _logsoftmax(v):
    return pl.pallas_call(
        _lsm_body,
        grid=(N // 1000,),
        in_specs=[pl.BlockSpec((1000, D), lambda i: (i, 0))],
        out_specs=pl.BlockSpec((1000, D), lambda i: (i, 0)),
        out_shape=jax.ShapeDtypeStruct((N, D), jnp.float32),
    )(v)


_MESH = plsc.VectorSubcoreMesh(core_axis_name="c", subcore_axis_name="s")


@functools.partial(
    pl.kernel,
    out_type=jax.ShapeDtypeStruct((N, D), jnp.float32),
    mesh=_MESH,
    scratch_types=[
        pltpu.VMEM((10, 2, CH), jnp.int32),
        pltpu.VMEM((5, CH, DH), jnp.float32),
        pltpu.VMEM_SHARED((N + 8, DH), jnp.float32),
        pltpu.VMEM_SHARED((N + 8, DH), jnp.float32),
        pltpu.SemaphoreType.DMA((10,)),
        pltpu.SemaphoreType.DMA((5,)),
        pltpu.SemaphoreType.DMA((5,)),
    ],
    compiler_params=pltpu.CompilerParams(use_tc_tiling_on_sc=False),
)
def _propagate(h_hbm, hs_hbm, idx_hbm, out_hbm,
               ir, gb, buf_a, buf_b, si, sg, ss):
    c = lax.axis_index("c")
    s = lax.axis_index("s")
    row0 = s * RPT
    col0 = c * DH

    def stripe_init(dst_buf, k):
        if k < 0:
            src = h_hbm.at[pl.ds(row0, RPT), pl.ds(col0, DH)]
        else:
            src = hs_hbm.at[k, pl.ds(row0, RPT), pl.ds(col0, DH)]
        pltpu.sync_copy(src, dst_buf.at[pl.ds(row0, RPT), :])

    def idx_issue(t, p):
        pltpu.async_copy(idx_hbm.at[s, t], ir.at[p], si.at[p])

    def idx_wait(t, p):
        pltpu.make_async_copy(idx_hbm.at[s, t], ir.at[p], si.at[p]).wait()

    def gather_issue(table, p, b, g):
        pltpu.async_copy(table.at[ir.at[p, 0]], gb.at[b], sg.at[g])

    def gather_wait(table, p, b, g):
        pltpu.make_async_copy(table.at[ir.at[p, 0]], gb.at[b],
                              sg.at[g]).wait()

    def scatter_issue(accum, p, b, w):
        pltpu.async_copy(gb.at[b], accum.at[ir.at[p, 1]], ss.at[w], add=True)

    def scatter_wait(accum, p, b, w):
        pltpu.make_async_copy(gb.at[b], accum.at[ir.at[p, 1]],
                              ss.at[w]).wait()

    def slot(table, accum, t, r, first_block, last_block):
        gather_wait(table, r, r % 5, r % 5)
        scatter_issue(accum, r, r % 5, r % 5)
        if not (first_block and r < 2):
            scatter_wait(accum, (r - 2) % 10, (r - 2) % 5, (r - 2) % 5)
        if not (last_block and r >= 7):
            idx_wait(t + 3, (r + 3) % 10)
            gather_issue(table, (r + 3) % 10, (r + 3) % 5, (r + 3) % 5)
        if not (last_block and r >= 5):
            idx_issue(t + 5, (r + 5) % 10)

    def phase2(table, accum):
        for p in range(5):
            idx_issue(p, p)
        for p in range(3):
            idx_wait(p, p)
            gather_issue(table, p, p, p)
        for r in range(10):
            slot(table, accum, r, r, True, False)

        @pl.loop(1, NCHUNK // 10 - 1)
        def _(j):
            t0 = j * 10
            for r in range(10):
                slot(table, accum, t0 + r, r, False, False)

        t0 = NCHUNK - 10
        for r in range(10):
            slot(table, accum, t0 + r, r, False, True)
        scatter_wait(accum, 8, 3, 3)
        scatter_wait(accum, 9, 4, 4)

    stripe_init(buf_a, -1)

    bufs = (buf_a, buf_b)
    for k in range(K):
        table = bufs[k % 2]
        accum = bufs[(k + 1) % 2]
        stripe_init(accum, k)
        plsc.subcore_barrier()
        phase2(table, accum)
        plsc.subcore_barrier()

    final = bufs[K % 2]
    pltpu.sync_copy(final.at[pl.ds(row0, RPT), :],
                    out_hbm.at[pl.ds(row0, RPT), pl.ds(col0, DH)])


def kernel(x, edge_index, W, b):
    h, hs = _linear(x, W, b.reshape(1, D))
    pad = NS * EPT - E
    src = jnp.concatenate(
        [edge_index[0], jnp.zeros((pad,), jnp.int32)]).reshape(NS, NCHUNK, 1, CH)
    dst = jnp.concatenate(
        [edge_index[1], jnp.full((pad,), N, jnp.int32)]).reshape(NS, NCHUNK, 1, CH)
    idx = jnp.concatenate([src, dst], axis=2)
    v = _propagate(h, hs, idx)
    return _logsoftmax(v)
